# Initial kernel scaffold; baseline (speedup 1.0000x reference)
import functools

import jax
import jax.numpy as jnp
from jax import lax
from jax.experimental import pallas as pl
from jax.experimental.pallas import tpu as pltpu
from jax.experimental.pallas import tpu_sc as plsc

_N = 10000
_E = 320000
_D_IN = 128
_H = 300
_HALF = 150
_HP = 160
_HF = 320
_BR = 1000

_NC = 2
_NS = 16
_CH = 128
_EPT = _E // _NS
_NFULL = _EPT // _CH
_TAIL = _EPT - _NFULL * _CH
_RPT = _N // _NS
_ZCH = 125


def _pad_cols(a):
    z = jnp.zeros((a.shape[0], _HP - _HALF), a.dtype)
    return jnp.concatenate([a[:, :_HALF], z, a[:, _HALF:], z], axis=1)


def _pad_rows(a):
    z = jnp.zeros((_HP - _HALF, a.shape[1]), a.dtype)
    return jnp.concatenate([a[:_HALF], z, a[_HALF:], z], axis=0)


def _pad_vec(v):
    z = jnp.zeros((_HP - _HALF,), v.dtype)
    return jnp.concatenate([v[:_HALF], z, v[_HALF:], z])


def _prep_gru(Wih, Whh, bih, bhh):
    wa, wb, wh, bi, bh = [], [], [], [], []
    for g in range(3):
        wi = _pad_rows(_pad_cols(Wih[g * _H:(g + 1) * _H, :].T))
        wa.append(wi[:_HP])
        wb.append(wi[_HP:])
        wh.append(_pad_rows(_pad_cols(Whh[g * _H:(g + 1) * _H, :].T)))
        bi.append(_pad_vec(bih[g * _H:(g + 1) * _H]))
        bh.append(_pad_vec(bhh[g * _H:(g + 1) * _H]))
    return (jnp.stack(wa), jnp.stack(wb), jnp.stack(wh),
            jnp.stack(bi)[:, None, :], jnp.stack(bh)[:, None, :])


def _mm_halves(h, wm0, wm1):
    def body(h_ref, w0_ref, w1_ref, o0_ref, o1_ref):
        hb = h_ref[...]
        o0_ref[...] = jnp.dot(hb, w0_ref[...], preferred_element_type=jnp.float32)
        o1_ref[...] = jnp.dot(hb, w1_ref[...], preferred_element_type=jnp.float32)

    return pl.pallas_call(
        body,
        grid=(_N // _BR,),
        in_specs=[
            pl.BlockSpec((_BR, _HF), lambda i: (i, 0)),
            pl.BlockSpec((_HF, _HP), lambda i: (0, 0)),
            pl.BlockSpec((_HF, _HP), lambda i: (0, 0)),
        ],
        out_specs=[pl.BlockSpec((_BR, _HP), lambda i: (i, 0))] * 2,
        out_shape=[jax.ShapeDtypeStruct((_N, _HP), jnp.float32)] * 2,
    )(h, wm0, wm1)


def _segment_sum_sc(m0, m1, edges, zblk):
    mesh = plsc.VectorSubcoreMesh(core_axis_name="c", subcore_axis_name="s",
                                  num_cores=_NC, num_subcores=_NS)

    @functools.partial(
        pl.kernel,
        out_type=[jax.ShapeDtypeStruct((_N, _HP), jnp.float32)] * 2,
        mesh=mesh,
        scratch_types=[
            pltpu.VMEM((_CH,), jnp.int32),
            pltpu.VMEM((_CH,), jnp.int32),
            pltpu.VMEM((_CH, _HP), jnp.float32),
            pltpu.VMEM((_TAIL,), jnp.int32),
            pltpu.VMEM((_TAIL,), jnp.int32),
            pltpu.VMEM((_TAIL, _HP), jnp.float32),
            pltpu.VMEM_SHARED((_N, _HP), jnp.float32),
        ],
    )
    def seg(m0_hbm, m1_hbm, edges_hbm, zeros_hbm, o0_hbm, o1_hbm,
            src_v, dst_v, rows_v, src_t, dst_t, rows_t, agg_sh):
        c = lax.axis_index("c")
        s = lax.axis_index("s")
        row0 = s * _RPT

        pltpu.sync_copy(zeros_hbm, rows_v)
        for k in range(_RPT // _ZCH):
            pltpu.sync_copy(rows_v.at[pl.ds(0, _ZCH)],
                            agg_sh.at[pl.ds(row0 + k * _ZCH, _ZCH)])
        plsc.subcore_barrier()

        ebase = s * _EPT

        def chunk(m_hbm, base, n, sv, dv, rv):
            pltpu.sync_copy(edges_hbm.at[0, pl.ds(base, n)], sv)
            pltpu.sync_copy(edges_hbm.at[1, pl.ds(base, n)], dv)
            pltpu.sync_copy(m_hbm.at[sv], rv)
            pltpu.sync_copy(rv, agg_sh.at[dv], add=True)

        def run(m_hbm):
            def body(i, carry):
                chunk(m_hbm, ebase + i * _CH, _CH, src_v, dst_v, rows_v)
                return carry
            lax.fori_loop(0, _NFULL, body, 0)
            chunk(m_hbm, ebase + _NFULL * _CH, _TAIL, src_t, dst_t, rows_t)

        @pl.when(c == 0)
        def _():
            run(m0_hbm)

        @pl.when(c == 1)
        def _():
            run(m1_hbm)

        plsc.subcore_barrier()

        @pl.when(c == 0)
        def _():
            pltpu.sync_copy(agg_sh.at[pl.ds(row0, _RPT)],
                            o0_hbm.at[pl.ds(row0, _RPT)])

        @pl.when(c == 1)
        def _():
            pltpu.sync_copy(agg_sh.at[pl.ds(row0, _RPT)],
                            o1_hbm.at[pl.ds(row0, _RPT)])

    return seg(m0, m1, edges, zblk)


def _gru(a0, a1, h, WA, WB, Wh, bi, bh, relu):
    def body(a0_ref, a1_ref, h_ref, wa_ref, wb_ref, wh_ref, bi_ref, bh_ref, o_ref):
        A0 = a0_ref[...]
        A1 = a1_ref[...]
        Hb = h_ref[...]

        def gate(g):
            xg = (jnp.dot(A0, wa_ref[g], preferred_element_type=jnp.float32)
                  + jnp.dot(A1, wb_ref[g], preferred_element_type=jnp.float32)
                  + bi_ref[g])
            hg = jnp.dot(Hb, wh_ref[g], preferred_element_type=jnp.float32) + bh_ref[g]
            return xg, hg

        xr, hr = gate(0)
        xz, hz = gate(1)
        xn, hn = gate(2)
        r = jax.nn.sigmoid(xr + hr)
        z = jax.nn.sigmoid(xz + hz)
        n = jnp.tanh(xn + r * hn)
        out = (1.0 - z) * n + z * Hb
        if relu:
            out = jnp.maximum(out, 0.0)
        o_ref[...] = out

    return pl.pallas_call(
        body,
        grid=(_N // _BR,),
        in_specs=[
            pl.BlockSpec((_BR, _HP), lambda i: (i, 0)),
            pl.BlockSpec((_BR, _HP), lambda i: (i, 0)),
            pl.BlockSpec((_BR, _HF), lambda i: (i, 0)),
            pl.BlockSpec((3, _HP, _HF), lambda i: (0, 0, 0)),
            pl.BlockSpec((3, _HP, _HF), lambda i: (0, 0, 0)),
            pl.BlockSpec((3, _HF, _HF), lambda i: (0, 0, 0)),
            pl.BlockSpec((3, 1, _HF), lambda i: (0, 0, 0)),
            pl.BlockSpec((3, 1, _HF), lambda i: (0, 0, 0)),
        ],
        out_specs=pl.BlockSpec((_BR, _HF), lambda i: (i, 0)),
        out_shape=jax.ShapeDtypeStruct((_N, _HF), jnp.float32),
    )(a0, a1, h, WA, WB, Wh, bi, bh)


def _mean_logsoftmax(h):
    nsteps = _N // _BR

    def body(h_ref, o_ref):
        i = pl.program_id(0)
        colsum = jnp.sum(h_ref[...], axis=0, keepdims=True)

        @pl.when(i == 0)
        def _():
            o_ref[...] = colsum

        @pl.when(i > 0)
        def _():
            o_ref[...] = o_ref[...] + colsum

        @pl.when(i == nsteps - 1)
        def _():
            g = o_ref[...] / float(_N)
            col = lax.broadcasted_iota(jnp.int32, (1, _HF), 1)
            valid = (col < _HALF) | ((col >= _HP) & (col < _HP + _HALF))
            gm = jnp.where(valid, g, -jnp.inf)
            mx = jnp.max(gm)
            se = jnp.sum(jnp.where(valid, jnp.exp(g - mx), 0.0))
            o_ref[...] = g - mx - jnp.log(se)

    return pl.pallas_call(
        body,
        grid=(nsteps,),
        in_specs=[pl.BlockSpec((_BR, _HF), lambda i: (i, 0))],
        out_specs=pl.BlockSpec((1, _HF), lambda i: (0, 0)),
        out_shape=jax.ShapeDtypeStruct((1, _HF), jnp.float32),
    )(h)


def kernel(x, edge_index, W1, Wih1, Whh1, bih1, bhh1,
           W2, Wih2, Whh2, bih2, bhh2):
    edges = edge_index.astype(jnp.int32)
    zblk = jnp.zeros((_CH, _HP), jnp.float32)

    h = jnp.zeros((_N, _HF), jnp.float32).at[:, :_D_IN].set(x)

    for (W, Wih, Whh, bih, bhh, last_relu) in (
            (W1, Wih1, Whh1, bih1, bhh1, True),
            (W2, Wih2, Whh2, bih2, bhh2, False)):
        WA, WB, Wh, bi, bh = _prep_gru(Wih, Whh, bih, bhh)
        for step in range(W.shape[0]):
            wm = _pad_rows(_pad_cols(W[step]))
            m0, m1 = _mm_halves(h, wm[:, :_HP], wm[:, _HP:])
            a0, a1 = _segment_sum_sc(m0, m1, edges, zblk)
            relu = last_relu and step == W.shape[0] - 1
            h = _gru(a0, a1, h, WA, WB, Wh, bi, bh, relu)

    o = _mean_logsoftmax(h)
    out = jnp.concatenate([o[0, :_HALF], o[0, _HP:_HP + _HALF]])
    return out.reshape(1, -1)


# speedup vs baseline: 3.5850x; 3.5850x over previous
import functools

import jax
import jax.numpy as jnp
from jax import lax
from jax.experimental import pallas as pl
from jax.experimental.pallas import tpu as pltpu
from jax.experimental.pallas import tpu_sc as plsc

_N = 10000
_E = 320000
_D_IN = 128
_H = 300
_HALF = 150
_HP = 160
_HF = 320
_BR = 1000

_NC = 2
_NS = 16
_CH = 128
_EPT = _E // _NS
_NFULL = _EPT // _CH
_TAIL = _EPT - _NFULL * _CH
_RPT = 640
_RPT_LAST = _N - 15 * _RPT
_ZCH = 80


def _pad_cols(a):
    z = jnp.zeros((a.shape[0], _HP - _HALF), a.dtype)
    return jnp.concatenate([a[:, :_HALF], z, a[:, _HALF:], z], axis=1)


def _pad_rows(a):
    z = jnp.zeros((_HP - _HALF, a.shape[1]), a.dtype)
    return jnp.concatenate([a[:_HALF], z, a[_HALF:], z], axis=0)


def _pad_vec(v):
    z = jnp.zeros((_HP - _HALF,), v.dtype)
    return jnp.concatenate([v[:_HALF], z, v[_HALF:], z])


def _prep_gru(Wih, Whh, bih, bhh):
    wa, wb, wh, bi, bh = [], [], [], [], []
    for g in range(3):
        wi = _pad_rows(_pad_cols(Wih[g * _H:(g + 1) * _H, :].T))
        wa.append(wi[:_HP])
        wb.append(wi[_HP:])
        wh.append(_pad_rows(_pad_cols(Whh[g * _H:(g + 1) * _H, :].T)))
        bi.append(_pad_vec(bih[g * _H:(g + 1) * _H]))
        bh.append(_pad_vec(bhh[g * _H:(g + 1) * _H]))
    return (jnp.stack(wa), jnp.stack(wb), jnp.stack(wh),
            jnp.stack(bi)[:, None, :], jnp.stack(bh)[:, None, :])


def _mm_halves(h, wm0, wm1):
    def body(h_ref, w0_ref, w1_ref, o0_ref, o1_ref):
        hb = h_ref[...]
        o0_ref[...] = jnp.dot(hb, w0_ref[...], preferred_element_type=jnp.float32)
        o1_ref[...] = jnp.dot(hb, w1_ref[...], preferred_element_type=jnp.float32)

    return pl.pallas_call(
        body,
        grid=(_N // _BR,),
        in_specs=[
            pl.BlockSpec((_BR, _HF), lambda i: (i, 0)),
            pl.BlockSpec((_HF, _HP), lambda i: (0, 0)),
            pl.BlockSpec((_HF, _HP), lambda i: (0, 0)),
        ],
        out_specs=[pl.BlockSpec((_BR, _HP), lambda i: (i, 0))] * 2,
        out_shape=[jax.ShapeDtypeStruct((_N, _HP), jnp.float32)] * 2,
    )(h, wm0, wm1)


def _segment_sum_sc(m0, m1, src, dst, zblk):
    mesh = plsc.VectorSubcoreMesh(core_axis_name="c", subcore_axis_name="s",
                                  num_cores=_NC, num_subcores=_NS)

    @functools.partial(
        pl.kernel,
        out_type=[jax.ShapeDtypeStruct((_N, _HP), jnp.float32)] * 2,
        mesh=mesh,
        compiler_params=pltpu.CompilerParams(use_tc_tiling_on_sc=False),
        scratch_types=[
            pltpu.VMEM((_CH,), jnp.int32),
            pltpu.VMEM((_CH,), jnp.int32),
            pltpu.VMEM((_CH, _HP), jnp.float32),
            pltpu.VMEM((_TAIL,), jnp.int32),
            pltpu.VMEM((_TAIL,), jnp.int32),
            pltpu.VMEM((_TAIL, _HP), jnp.float32),
            pltpu.VMEM_SHARED((_N, _HP), jnp.float32),
        ],
    )
    def seg(m0_hbm, m1_hbm, src_hbm, dst_hbm, zeros_hbm, o0_hbm, o1_hbm,
            src_v, dst_v, rows_v, src_t, dst_t, rows_t, agg_sh):
        c = lax.axis_index("c")
        s = lax.axis_index("s")
        row0 = s * _RPT

        pltpu.sync_copy(zeros_hbm, rows_v)

        @pl.when(s < _NS - 1)
        def _():
            for k in range(_RPT // _ZCH):
                pltpu.sync_copy(rows_v.at[pl.ds(0, _ZCH)],
                                agg_sh.at[pl.ds(row0 + k * _ZCH, _ZCH)])

        @pl.when(s == _NS - 1)
        def _():
            for k in range(_RPT_LAST // _ZCH):
                pltpu.sync_copy(rows_v.at[pl.ds(0, _ZCH)],
                                agg_sh.at[pl.ds(row0 + k * _ZCH, _ZCH)])

        plsc.subcore_barrier()

        ebase = s * _EPT

        def chunk(m_hbm, base, n, sv, dv, rv):
            pltpu.sync_copy(src_hbm.at[pl.ds(base, n)], sv)
            pltpu.sync_copy(dst_hbm.at[pl.ds(base, n)], dv)
            pltpu.sync_copy(m_hbm.at[sv], rv)
            pltpu.sync_copy(rv, agg_sh.at[dv], add=True)

        def run(m_hbm):
            def body(i, carry):
                chunk(m_hbm, ebase + i * _CH, _CH, src_v, dst_v, rows_v)
                return carry
            lax.fori_loop(0, _NFULL, body, 0)
            chunk(m_hbm, ebase + _NFULL * _CH, _TAIL, src_t, dst_t, rows_t)

        @pl.when(c == 0)
        def _():
            run(m0_hbm)

        @pl.when(c == 1)
        def _():
            run(m1_hbm)

        plsc.subcore_barrier()

        def wout(o_hbm):
            @pl.when(s < _NS - 1)
            def _():
                pltpu.sync_copy(agg_sh.at[pl.ds(row0, _RPT)],
                                o_hbm.at[pl.ds(row0, _RPT)])

            @pl.when(s == _NS - 1)
            def _():
                pltpu.sync_copy(agg_sh.at[pl.ds(row0, _RPT_LAST)],
                                o_hbm.at[pl.ds(row0, _RPT_LAST)])

        @pl.when(c == 0)
        def _():
            wout(o0_hbm)

        @pl.when(c == 1)
        def _():
            wout(o1_hbm)

    return seg(m0, m1, src, dst, zblk)


def _gru(a0, a1, h, WA, WB, Wh, bi, bh, relu):
    def body(a0_ref, a1_ref, h_ref, wa_ref, wb_ref, wh_ref, bi_ref, bh_ref, o_ref):
        A0 = a0_ref[...]
        A1 = a1_ref[...]
        Hb = h_ref[...]

        def gate(g):
            xg = (jnp.dot(A0, wa_ref[g], preferred_element_type=jnp.float32)
                  + jnp.dot(A1, wb_ref[g], preferred_element_type=jnp.float32)
                  + bi_ref[g])
            hg = jnp.dot(Hb, wh_ref[g], preferred_element_type=jnp.float32) + bh_ref[g]
            return xg, hg

        xr, hr = gate(0)
        xz, hz = gate(1)
        xn, hn = gate(2)
        r = jax.nn.sigmoid(xr + hr)
        z = jax.nn.sigmoid(xz + hz)
        n = jnp.tanh(xn + r * hn)
        out = (1.0 - z) * n + z * Hb
        if relu:
            out = jnp.maximum(out, 0.0)
        o_ref[...] = out

    return pl.pallas_call(
        body,
        grid=(_N // _BR,),
        in_specs=[
            pl.BlockSpec((_BR, _HP), lambda i: (i, 0)),
            pl.BlockSpec((_BR, _HP), lambda i: (i, 0)),
            pl.BlockSpec((_BR, _HF), lambda i: (i, 0)),
            pl.BlockSpec((3, _HP, _HF), lambda i: (0, 0, 0)),
            pl.BlockSpec((3, _HP, _HF), lambda i: (0, 0, 0)),
            pl.BlockSpec((3, _HF, _HF), lambda i: (0, 0, 0)),
            pl.BlockSpec((3, 1, _HF), lambda i: (0, 0, 0)),
            pl.BlockSpec((3, 1, _HF), lambda i: (0, 0, 0)),
        ],
        out_specs=pl.BlockSpec((_BR, _HF), lambda i: (i, 0)),
        out_shape=jax.ShapeDtypeStruct((_N, _HF), jnp.float32),
    )(a0, a1, h, WA, WB, Wh, bi, bh)


def _mean_logsoftmax(h):
    nsteps = _N // _BR

    def body(h_ref, o_ref):
        i = pl.program_id(0)
        colsum = jnp.sum(h_ref[...], axis=0, keepdims=True)

        @pl.when(i == 0)
        def _():
            o_ref[...] = colsum

        @pl.when(i > 0)
        def _():
            o_ref[...] = o_ref[...] + colsum

        @pl.when(i == nsteps - 1)
        def _():
            g = o_ref[...] / float(_N)
            col = lax.broadcasted_iota(jnp.int32, (1, _HF), 1)
            valid = (col < _HALF) | ((col >= _HP) & (col < _HP + _HALF))
            gm = jnp.where(valid, g, -jnp.inf)
            mx = jnp.max(gm)
            se = jnp.sum(jnp.where(valid, jnp.exp(g - mx), 0.0))
            o_ref[...] = g - mx - jnp.log(se)

    return pl.pallas_call(
        body,
        grid=(nsteps,),
        in_specs=[pl.BlockSpec((_BR, _HF), lambda i: (i, 0))],
        out_specs=pl.BlockSpec((1, _HF), lambda i: (0, 0)),
        out_shape=jax.ShapeDtypeStruct((1, _HF), jnp.float32),
    )(h)


def kernel(x, edge_index, W1, Wih1, Whh1, bih1, bhh1,
           W2, Wih2, Whh2, bih2, bhh2):
    edges = edge_index.astype(jnp.int32)
    src_ix, dst_ix = edges[0], edges[1]
    zblk = jnp.zeros((_CH, _HP), jnp.float32)

    h = jnp.zeros((_N, _HF), jnp.float32).at[:, :_D_IN].set(x)

    for (W, Wih, Whh, bih, bhh, last_relu) in (
            (W1, Wih1, Whh1, bih1, bhh1, True),
            (W2, Wih2, Whh2, bih2, bhh2, False)):
        WA, WB, Wh, bi, bh = _prep_gru(Wih, Whh, bih, bhh)
        for step in range(W.shape[0]):
            wm = _pad_rows(_pad_cols(W[step]))
            m0, m1 = _mm_halves(h, wm[:, :_HP], wm[:, _HP:])
            a0, a1 = _segment_sum_sc(m0, m1, src_ix, dst_ix, zblk)
            relu = last_relu and step == W.shape[0] - 1
            h = _gru(a0, a1, h, WA, WB, Wh, bi, bh, relu)

    o = _mean_logsoftmax(h)
    out = jnp.concatenate([o[0, :_HALF], o[0, _HP:_HP + _HALF]])
    return out.reshape(1, -1)


# speedup vs baseline: 5.1764x; 1.4439x over previous
import functools

import jax
import jax.numpy as jnp
from jax import lax
from jax.experimental import pallas as pl
from jax.experimental.pallas import tpu as pltpu
from jax.experimental.pallas import tpu_sc as plsc

_N = 10000
_E = 320000
_D_IN = 128
_H = 300
_HALF = 150
_HP = 160
_HF = 320
_BR = 1000

_NC = 2
_NS = 16
_CH = 80
_NCH = _E // _CH
_CPT = _NCH // _NS
_RPT = 640
_RPT_LAST = _N - 15 * _RPT
_ZCH = 80


def _pad_cols(a):
    z = jnp.zeros((a.shape[0], _HP - _HALF), a.dtype)
    return jnp.concatenate([a[:, :_HALF], z, a[:, _HALF:], z], axis=1)


def _pad_rows(a):
    z = jnp.zeros((_HP - _HALF, a.shape[1]), a.dtype)
    return jnp.concatenate([a[:_HALF], z, a[_HALF:], z], axis=0)


def _pad_vec(v):
    z = jnp.zeros((_HP - _HALF,), v.dtype)
    return jnp.concatenate([v[:_HALF], z, v[_HALF:], z])


def _prep_gru(Wih, Whh, bih, bhh):
    wa, wb, wh, bi, bh = [], [], [], [], []
    for g in range(3):
        wi = _pad_rows(_pad_cols(Wih[g * _H:(g + 1) * _H, :].T))
        wa.append(wi[:_HP])
        wb.append(wi[_HP:])
        wh.append(_pad_rows(_pad_cols(Whh[g * _H:(g + 1) * _H, :].T)))
        bi.append(_pad_vec(bih[g * _H:(g + 1) * _H]))
        bh.append(_pad_vec(bhh[g * _H:(g + 1) * _H]))
    return (jnp.stack(wa), jnp.stack(wb), jnp.stack(wh),
            jnp.stack(bi)[:, None, :], jnp.stack(bh)[:, None, :])


def _mm_halves(h, wm0, wm1):
    def body(h_ref, w0_ref, w1_ref, o0_ref, o1_ref):
        hb = h_ref[...]
        o0_ref[...] = jnp.dot(hb, w0_ref[...], preferred_element_type=jnp.float32)
        o1_ref[...] = jnp.dot(hb, w1_ref[...], preferred_element_type=jnp.float32)

    return pl.pallas_call(
        body,
        grid=(_N // _BR,),
        in_specs=[
            pl.BlockSpec((_BR, _HF), lambda i: (i, 0)),
            pl.BlockSpec((_HF, _HP), lambda i: (0, 0)),
            pl.BlockSpec((_HF, _HP), lambda i: (0, 0)),
        ],
        out_specs=[pl.BlockSpec((_BR, _HP), lambda i: (i, 0))] * 2,
        out_shape=[jax.ShapeDtypeStruct((_N, _HP), jnp.float32)] * 2,
    )(h, wm0, wm1)


def _segment_sum_sc(m0, m1, src, dst, zblk):
    mesh = plsc.VectorSubcoreMesh(core_axis_name="c", subcore_axis_name="s",
                                  num_cores=_NC, num_subcores=_NS)

    @functools.partial(
        pl.kernel,
        out_type=[jax.ShapeDtypeStruct((_N, _HP), jnp.float32)] * 2,
        mesh=mesh,
        compiler_params=pltpu.CompilerParams(use_tc_tiling_on_sc=False),
        scratch_types=[
            pltpu.VMEM((_CH,), jnp.int32),
            pltpu.VMEM((_CH,), jnp.int32),
            pltpu.VMEM((_CH,), jnp.int32),
            pltpu.VMEM((_CH,), jnp.int32),
            pltpu.VMEM((_CH, _HP), jnp.float32),
            pltpu.VMEM((_CH, _HP), jnp.float32),
            pltpu.VMEM_SHARED((_N, _HP), jnp.float32),
            pltpu.SemaphoreType.DMA,
            pltpu.SemaphoreType.DMA,
            pltpu.SemaphoreType.DMA,
            pltpu.SemaphoreType.DMA,
        ],
    )
    def seg(m0_hbm, m1_hbm, src_hbm, dst_hbm, zeros_hbm, o0_hbm, o1_hbm,
            sv0, dv0, sv1, dv1, rv0, rv1, agg_sh,
            sem_g0, sem_g1, sem_i0, sem_i1):
        c = lax.axis_index("c")
        s = lax.axis_index("s")
        row0 = s * _RPT

        pltpu.sync_copy(zeros_hbm, rv0)

        @pl.when(s < _NS - 1)
        def _():
            for k in range(_RPT // _ZCH):
                pltpu.sync_copy(rv0.at[pl.ds(0, _ZCH)],
                                agg_sh.at[pl.ds(row0 + k * _ZCH, _ZCH)])

        @pl.when(s == _NS - 1)
        def _():
            for k in range(_RPT_LAST // _ZCH):
                pltpu.sync_copy(rv0.at[pl.ds(0, _ZCH)],
                                agg_sh.at[pl.ds(row0 + k * _ZCH, _ZCH)])

        plsc.subcore_barrier()

        r0 = s * _CPT
        npairs = _CPT // 2

        def idx_load(r, sv, dv, sem):
            rr = jnp.minimum(r, _NCH - 1)
            pltpu.async_copy(src_hbm.at[rr], sv, sem)
            pltpu.async_copy(dst_hbm.at[rr], dv, sem)

        def idx_wait(sv, dv, sem):
            pltpu.make_async_copy(src_hbm.at[0], sv, sem).wait()
            pltpu.make_async_copy(dst_hbm.at[0], dv, sem).wait()

        def run(m_hbm):
            def gather_start(sv, rv, sem):
                pltpu.async_copy(m_hbm.at[sv], rv, sem)

            def gather_wait(sv, rv, sem):
                pltpu.make_async_copy(m_hbm.at[sv], rv, sem).wait()

            idx_load(r0, sv0, dv0, sem_i0)
            idx_load(r0 + 1, sv1, dv1, sem_i1)
            idx_wait(sv0, dv0, sem_i0)
            gather_start(sv0, rv0, sem_g0)

            def body(j, carry):
                a = r0 + 2 * j
                idx_wait(sv1, dv1, sem_i1)
                gather_wait(sv0, rv0, sem_g0)
                gather_start(sv1, rv1, sem_g1)
                pltpu.sync_copy(rv0, agg_sh.at[dv0], add=True)
                idx_load(a + 2, sv0, dv0, sem_i0)
                idx_wait(sv0, dv0, sem_i0)
                gather_wait(sv1, rv1, sem_g1)
                gather_start(sv0, rv0, sem_g0)
                pltpu.sync_copy(rv1, agg_sh.at[dv1], add=True)
                idx_load(a + 3, sv1, dv1, sem_i1)
                return carry

            lax.fori_loop(0, npairs, body, 0)
            gather_wait(sv0, rv0, sem_g0)
            idx_wait(sv1, dv1, sem_i1)

        @pl.when(c == 0)
        def _():
            run(m0_hbm)

        @pl.when(c == 1)
        def _():
            run(m1_hbm)

        plsc.subcore_barrier()

        def wout(o_hbm):
            @pl.when(s < _NS - 1)
            def _():
                pltpu.sync_copy(agg_sh.at[pl.ds(row0, _RPT)],
                                o_hbm.at[pl.ds(row0, _RPT)])

            @pl.when(s == _NS - 1)
            def _():
                pltpu.sync_copy(agg_sh.at[pl.ds(row0, _RPT_LAST)],
                                o_hbm.at[pl.ds(row0, _RPT_LAST)])

        @pl.when(c == 0)
        def _():
            wout(o0_hbm)

        @pl.when(c == 1)
        def _():
            wout(o1_hbm)

    return seg(m0, m1, src, dst, zblk)


def _gru(a0, a1, h, WA, WB, Wh, bi, bh, relu):
    def body(a0_ref, a1_ref, h_ref, wa_ref, wb_ref, wh_ref, bi_ref, bh_ref, o_ref):
        A0 = a0_ref[...]
        A1 = a1_ref[...]
        Hb = h_ref[...]

        def gate(g):
            xg = (jnp.dot(A0, wa_ref[g], preferred_element_type=jnp.float32)
                  + jnp.dot(A1, wb_ref[g], preferred_element_type=jnp.float32)
                  + bi_ref[g])
            hg = jnp.dot(Hb, wh_ref[g], preferred_element_type=jnp.float32) + bh_ref[g]
            return xg, hg

        xr, hr = gate(0)
        xz, hz = gate(1)
        xn, hn = gate(2)
        r = jax.nn.sigmoid(xr + hr)
        z = jax.nn.sigmoid(xz + hz)
        n = jnp.tanh(xn + r * hn)
        out = (1.0 - z) * n + z * Hb
        if relu:
            out = jnp.maximum(out, 0.0)
        o_ref[...] = out

    return pl.pallas_call(
        body,
        grid=(_N // _BR,),
        in_specs=[
            pl.BlockSpec((_BR, _HP), lambda i: (i, 0)),
            pl.BlockSpec((_BR, _HP), lambda i: (i, 0)),
            pl.BlockSpec((_BR, _HF), lambda i: (i, 0)),
            pl.BlockSpec((3, _HP, _HF), lambda i: (0, 0, 0)),
            pl.BlockSpec((3, _HP, _HF), lambda i: (0, 0, 0)),
            pl.BlockSpec((3, _HF, _HF), lambda i: (0, 0, 0)),
            pl.BlockSpec((3, 1, _HF), lambda i: (0, 0, 0)),
            pl.BlockSpec((3, 1, _HF), lambda i: (0, 0, 0)),
        ],
        out_specs=pl.BlockSpec((_BR, _HF), lambda i: (i, 0)),
        out_shape=jax.ShapeDtypeStruct((_N, _HF), jnp.float32),
    )(a0, a1, h, WA, WB, Wh, bi, bh)


def _mean_logsoftmax(h):
    nsteps = _N // _BR

    def body(h_ref, o_ref):
        i = pl.program_id(0)
        colsum = jnp.sum(h_ref[...], axis=0, keepdims=True)

        @pl.when(i == 0)
        def _():
            o_ref[...] = colsum

        @pl.when(i > 0)
        def _():
            o_ref[...] = o_ref[...] + colsum

        @pl.when(i == nsteps - 1)
        def _():
            g = o_ref[...] / float(_N)
            col = lax.broadcasted_iota(jnp.int32, (1, _HF), 1)
            valid = (col < _HALF) | ((col >= _HP) & (col < _HP + _HALF))
            gm = jnp.where(valid, g, -jnp.inf)
            mx = jnp.max(gm)
            se = jnp.sum(jnp.where(valid, jnp.exp(g - mx), 0.0))
            o_ref[...] = g - mx - jnp.log(se)

    return pl.pallas_call(
        body,
        grid=(nsteps,),
        in_specs=[pl.BlockSpec((_BR, _HF), lambda i: (i, 0))],
        out_specs=pl.BlockSpec((1, _HF), lambda i: (0, 0)),
        out_shape=jax.ShapeDtypeStruct((1, _HF), jnp.float32),
    )(h)


def kernel(x, edge_index, W1, Wih1, Whh1, bih1, bhh1,
           W2, Wih2, Whh2, bih2, bhh2):
    edges = edge_index.astype(jnp.int32)
    src_ix = edges[0].reshape(_NCH, _CH)
    dst_ix = edges[1].reshape(_NCH, _CH)
    zblk = jnp.zeros((_CH, _HP), jnp.float32)

    h = jnp.zeros((_N, _HF), jnp.float32).at[:, :_D_IN].set(x)

    for (W, Wih, Whh, bih, bhh, last_relu) in (
            (W1, Wih1, Whh1, bih1, bhh1, True),
            (W2, Wih2, Whh2, bih2, bhh2, False)):
        WA, WB, Wh, bi, bh = _prep_gru(Wih, Whh, bih, bhh)
        for step in range(W.shape[0]):
            wm = _pad_rows(_pad_cols(W[step]))
            m0, m1 = _mm_halves(h, wm[:, :_HP], wm[:, _HP:])
            a0, a1 = _segment_sum_sc(m0, m1, src_ix, dst_ix, zblk)
            relu = last_relu and step == W.shape[0] - 1
            h = _gru(a0, a1, h, WA, WB, Wh, bi, bh, relu)

    o = _mean_logsoftmax(h)
    out = jnp.concatenate([o[0, :_HALF], o[0, _HP:_HP + _HALF]])
    return out.reshape(1, -1)
